# Initial kernel scaffold; baseline (speedup 1.0000x reference)
import jax
import jax.numpy as jnp
from jax.experimental import pallas as pl


def kernel(qlen, klen, rel_bias_table):
    raise NotImplementedError("write your pallas kernel here")


# speedup vs baseline: 42.8119x; 42.8119x over previous
import functools

import jax
import jax.numpy as jnp
import numpy as np
from jax import lax
from jax.experimental import pallas as pl
from jax.experimental.pallas import tpu as pltpu
from jax.experimental.pallas import tpu_sc as plsc

H = 16
S = 2048
NB = 32
SHIFTS = 16
PW = 4224
LW = 4352

_BLOCKS_PER_HEAD = S // SHIFTS
_LOG16 = np.float32(np.log(np.float64(16.0)))


def _line_tc_body(tt_ref, p_ref):
    u = lax.broadcasted_iota(jnp.int32, (H, LW), 1)
    d = u - (S - 1)
    neg = d < 0
    ad = jnp.abs(d)
    is_small = ad < 8
    x = ad.astype(jnp.float32) / jnp.float32(8.0)
    vlarge = 8 + (jnp.log(x) / _LOG16 * jnp.float32(8.0)).astype(jnp.int32)
    vlarge = jnp.minimum(vlarge, 15)
    bucket = jnp.where(neg, 16, 0) + jnp.where(is_small, ad, vlarge)
    line = jnp.zeros((H, LW), jnp.float32)
    for b in range(NB):
        line = jnp.where(bucket == b, tt_ref[:, b : b + 1], line)
    for r in range(SHIFTS):
        sh = SHIFTS - 1 - r
        p_ref[:, r, :] = lax.slice(line, (0, sh), (H, sh + PW))


_line_tc = pl.pallas_call(
    _line_tc_body,
    out_shape=jax.ShapeDtypeStruct((H, SHIFTS, PW), jnp.float32),
)


@functools.partial(
    pl.kernel,
    out_type=jax.ShapeDtypeStruct((H, S, S), jnp.float32),
    mesh=plsc.VectorSubcoreMesh(core_axis_name="c", subcore_axis_name="s"),
    scratch_types=[pltpu.VMEM((SHIFTS, PW), jnp.float32)],
    compiler_params=pltpu.CompilerParams(use_tc_tiling_on_sc=False),
)
def _expand_sc(p_hbm, out_hbm, p_v):
    info = plsc.get_sparse_core_info()
    nc = info.num_cores
    wid = lax.axis_index("s") * nc + lax.axis_index("c")
    h = wid // 2
    half = wid % 2
    pltpu.sync_copy(p_hbm.at[h], p_v)
    q0 = half * (_BLOCKS_PER_HEAD // 2)

    def body(q, carry):
        qq = q0 + q
        base = SHIFTS * (_BLOCKS_PER_HEAD - 1 - qq)
        pltpu.sync_copy(
            p_v.at[:, pl.ds(base, S)],
            out_hbm.at[h, pl.ds(SHIFTS * qq, SHIFTS)],
        )
        return carry

    lax.fori_loop(0, _BLOCKS_PER_HEAD // 2, body, 0)


def kernel(qlen, klen, rel_bias_table):
    tt = jnp.transpose(rel_bias_table)
    p_all = _line_tc(tt)
    out = _expand_sc(p_all)
    return out.reshape(1, H, S, S)


# speedup vs baseline: 42.8865x; 1.0017x over previous
import functools

import jax
import jax.numpy as jnp
import numpy as np
from jax import lax
from jax.experimental import pallas as pl
from jax.experimental.pallas import tpu as pltpu
from jax.experimental.pallas import tpu_sc as plsc

H = 16
S = 2048
NB = 32
SHIFTS = 16
PW = 4224
LW = 4352

_BLOCKS_PER_HEAD = S // SHIFTS
_LOG16 = np.float32(np.log(np.float64(16.0)))


def _line_tc_body(tt_ref, p_ref):
    u = lax.broadcasted_iota(jnp.int32, (H, LW), 1)
    d = u - (S - 1)
    neg = d < 0
    ad = jnp.abs(d)
    is_small = ad < 8
    x = ad.astype(jnp.float32) / jnp.float32(8.0)
    vlarge = 8 + (jnp.log(x) / _LOG16 * jnp.float32(8.0)).astype(jnp.int32)
    vlarge = jnp.minimum(vlarge, 15)
    bucket = jnp.where(neg, 16, 0) + jnp.where(is_small, ad, vlarge)
    line = jnp.zeros((H, LW), jnp.float32)
    for b in range(NB):
        line = jnp.where(bucket == b, tt_ref[:, b : b + 1], line)
    for r in range(SHIFTS):
        sh = SHIFTS - 1 - r
        p_ref[:, r, :] = lax.slice(line, (0, sh), (H, sh + PW))


_line_tc = pl.pallas_call(
    _line_tc_body,
    out_shape=jax.ShapeDtypeStruct((H, SHIFTS, PW), jnp.float32),
)


@functools.partial(
    pl.kernel,
    out_type=jax.ShapeDtypeStruct((1, H, S, S), jnp.float32),
    mesh=plsc.VectorSubcoreMesh(core_axis_name="c", subcore_axis_name="s"),
    scratch_types=[pltpu.VMEM((SHIFTS, PW), jnp.float32)],
    compiler_params=pltpu.CompilerParams(use_tc_tiling_on_sc=False),
)
def _expand_sc(p_hbm, out_hbm, p_v):
    info = plsc.get_sparse_core_info()
    nc = info.num_cores
    wid = lax.axis_index("s") * nc + lax.axis_index("c")
    h = wid // 2
    half = wid % 2
    pltpu.sync_copy(p_hbm.at[h], p_v)
    q0 = half * (_BLOCKS_PER_HEAD // 2)

    def body(q, carry):
        qq = q0 + q
        base = SHIFTS * (_BLOCKS_PER_HEAD - 1 - qq)
        pltpu.sync_copy(
            p_v.at[:, pl.ds(base, S)],
            out_hbm.at[0, h, pl.ds(SHIFTS * qq, SHIFTS)],
        )
        return carry

    lax.fori_loop(0, _BLOCKS_PER_HEAD // 2, body, 0)


def kernel(qlen, klen, rel_bias_table):
    tt = jnp.transpose(rel_bias_table)
    p_all = _line_tc(tt)
    return _expand_sc(p_all)
